# Initial kernel scaffold; baseline (speedup 1.0000x reference)
import jax
import jax.numpy as jnp
from jax.experimental import pallas as pl


def kernel(node_features, edge_index, W_l1, b_l1, W_r1, b_r1, W_l2, b_l2, W_r2, b_r2, W_p1, b_p1, W_p2, b_p2):
    raise NotImplementedError("write your pallas kernel here")


# speedup vs baseline: 4.9516x; 4.9516x over previous
import jax
import jax.numpy as jnp
from jax import lax
from jax.experimental import pallas as pl
from jax.experimental.pallas import tpu as pltpu
from jax.experimental.pallas import tpu_sc as plsc

N = 10000
E = 320000
D = 128

NC = 2
NS = 16
L = 16
NW = NC * NS

EPW = E // NW
CHUNK = 80
NCHUNK = EPW // CHUNK
NP = 10240
RPT = NP // NS
RZ = 64
CW = 16

_MESH = plsc.VectorSubcoreMesh(core_axis_name="c", subcore_axis_name="s")


def _sc_agg_body(x_hbm, src_hbm, dst_hbm, sum_out, sidx, didx, rows, zbuf,
                 sum_sh, sem):
    c = lax.axis_index("c")
    s = lax.axis_index("s")
    wid = s * NC + c
    base = wid * EPW

    zero = jnp.zeros((L,), jnp.float32)

    def zrow(i, _):
        for j in range(D // L):
            zbuf[i, pl.ds(j * L, L)] = zero
        return 0

    lax.fori_loop(0, RZ, zrow, 0)

    for j in range(RPT // RZ):
        pltpu.sync_copy(zbuf, sum_sh.at[pl.ds(s * RPT + j * RZ, RZ)])
    plsc.subcore_barrier()

    def chunk_body(k, _):
        off = base + k * CHUNK
        pltpu.sync_copy(src_hbm.at[pl.ds(off, CHUNK)], sidx)
        pltpu.sync_copy(dst_hbm.at[pl.ds(off, CHUNK)], didx)
        pltpu.async_copy(x_hbm.at[sidx], rows, sem).wait()
        pltpu.sync_copy(rows, sum_sh.at[didx], add=True)
        return 0

    lax.fori_loop(0, NCHUNK, chunk_body, 0)
    plsc.subcore_barrier()

    for j in range(RPT // RZ):
        r0 = s * RPT + j * RZ
        pltpu.sync_copy(sum_sh.at[pl.ds(r0, RZ)],
                        sum_out.at[pl.ds(c * NP + r0, RZ)])


_sc_agg = pl.kernel(
    _sc_agg_body,
    out_type=jax.ShapeDtypeStruct((NC * NP, D), jnp.float32),
    mesh=_MESH,
    scratch_types=[
        pltpu.VMEM((CHUNK,), jnp.int32),
        pltpu.VMEM((CHUNK,), jnp.int32),
        pltpu.VMEM((CHUNK, D), jnp.float32),
        pltpu.VMEM((RZ, D), jnp.float32),
        pltpu.VMEM_SHARED((NP, D), jnp.float32),
        pltpu.SemaphoreType.DMA,
    ],
)


def _sc_cnt_body(dst_hbm, cnt_out, didx, ones, z16, cnt_sh):
    c = lax.axis_index("c")
    s = lax.axis_index("s")
    wid = s * NC + c
    base = wid * EPW

    zero = jnp.zeros((L,), jnp.float32)

    def orow(i, _):
        ones[i, :] = zero + 1.0
        return 0

    def z16row(i, _):
        z16[i, :] = zero
        return 0

    lax.fori_loop(0, CHUNK, orow, 0)
    lax.fori_loop(0, RZ, z16row, 0)

    for j in range(RPT // RZ):
        pltpu.sync_copy(z16, cnt_sh.at[pl.ds(s * RPT + j * RZ, RZ)])
    plsc.subcore_barrier()

    def chunk_body(k, _):
        off = base + k * CHUNK
        pltpu.sync_copy(dst_hbm.at[pl.ds(off, CHUNK)], didx)
        pltpu.sync_copy(ones, cnt_sh.at[didx], add=True)
        return 0

    lax.fori_loop(0, NCHUNK, chunk_body, 0)
    plsc.subcore_barrier()

    for j in range(RPT // RZ):
        r0 = s * RPT + j * RZ
        pltpu.sync_copy(cnt_sh.at[pl.ds(r0, RZ)],
                        cnt_out.at[pl.ds(c * NP + r0, RZ)])


_sc_cnt = pl.kernel(
    _sc_cnt_body,
    out_type=jax.ShapeDtypeStruct((NC * NP, CW), jnp.float32),
    mesh=_MESH,
    scratch_types=[
        pltpu.VMEM((CHUNK,), jnp.int32),
        pltpu.VMEM((CHUNK, CW), jnp.float32),
        pltpu.VMEM((RZ, CW), jnp.float32),
        pltpu.VMEM_SHARED((NP, CW), jnp.float32),
    ],
)

BN = 1000


def _tc_layer1_body(x_ref, p0_ref, p1_ref, c0_ref, c1_ref, wl_ref, wr_ref,
                    b_ref, o_ref):
    summed = p0_ref[...] + p1_ref[...]
    cnt = c0_ref[...][:, 0:1] + c1_ref[...][:, 0:1]
    agg = summed / jnp.maximum(cnt, 1.0)
    h = (jnp.dot(x_ref[...], wl_ref[...], preferred_element_type=jnp.float32)
         + jnp.dot(agg, wr_ref[...], preferred_element_type=jnp.float32)
         + b_ref[0:1, :])
    nrm = jnp.sqrt(jnp.sum(h * h, axis=-1, keepdims=True))
    h = h / jnp.maximum(nrm, 1e-12)
    o_ref[...] = jnp.maximum(h, 0.0)


def _tc_layer2_body(x_ref, p0_ref, p1_ref, c0_ref, c1_ref, wl_ref, wr_ref,
                    b_ref, wp1_ref, bp1_ref, wp2_ref, bp2_ref, o_ref):
    summed = p0_ref[...] + p1_ref[...]
    cnt = c0_ref[...][:, 0:1] + c1_ref[...][:, 0:1]
    agg = summed / jnp.maximum(cnt, 1.0)
    h = (jnp.dot(x_ref[...], wl_ref[...], preferred_element_type=jnp.float32)
         + jnp.dot(agg, wr_ref[...], preferred_element_type=jnp.float32)
         + b_ref[0:1, :])
    nrm = jnp.sqrt(jnp.sum(h * h, axis=-1, keepdims=True))
    h = h / jnp.maximum(nrm, 1e-12)
    h = jnp.maximum(h, 0.0)
    g = jnp.maximum(
        jnp.dot(h, wp1_ref[...], preferred_element_type=jnp.float32)
        + bp1_ref[0:1, :], 0.0)
    o_ref[...] = (jnp.dot(g, wp2_ref[...], preferred_element_type=jnp.float32)
                  + bp2_ref[0:1, :])


def _row_spec(width):
    return pl.BlockSpec((BN, width), lambda i: (i, 0))


def _full_spec(r, cdim):
    return pl.BlockSpec((r, cdim), lambda i: (0, 0))


def _tc_layer1(x, p0, p1, c0, c1, wl, wr, bias):
    return pl.pallas_call(
        _tc_layer1_body,
        grid=(N // BN,),
        in_specs=[_row_spec(D), _row_spec(D), _row_spec(D), _row_spec(CW),
                  _row_spec(CW), _full_spec(D, D), _full_spec(D, D),
                  _full_spec(8, D)],
        out_specs=_row_spec(D),
        out_shape=jax.ShapeDtypeStruct((N, D), jnp.float32),
    )(x, p0, p1, c0, c1, wl, wr, bias)


def _tc_layer2(x, p0, p1, c0, c1, wl, wr, bias, wp1, bp1, wp2, bp2):
    return pl.pallas_call(
        _tc_layer2_body,
        grid=(N // BN,),
        in_specs=[_row_spec(D), _row_spec(D), _row_spec(D), _row_spec(CW),
                  _row_spec(CW), _full_spec(D, D), _full_spec(D, D),
                  _full_spec(8, D), _full_spec(D, D), _full_spec(8, D),
                  _full_spec(D, D), _full_spec(8, D)],
        out_specs=_row_spec(D),
        out_shape=jax.ShapeDtypeStruct((N, D), jnp.float32),
    )(x, p0, p1, c0, c1, wl, wr, bias, wp1, bp1, wp2, bp2)


def _pad8(b):
    return jnp.broadcast_to(b[None, :], (8, b.shape[0]))


def kernel(node_features, edge_index, W_l1, b_l1, W_r1, b_r1, W_l2, b_l2,
           W_r2, b_r2, W_p1, b_p1, W_p2, b_p2):
    src = edge_index[0]
    dst = edge_index[1]

    cnt = _sc_cnt(dst)
    sum1 = _sc_agg(node_features, src, dst)
    h1 = _tc_layer1(node_features, sum1[:N], sum1[NP:NP + N], cnt[:N],
                    cnt[NP:NP + N], W_l1, W_r1, _pad8(b_l1 + b_r1))

    sum2 = _sc_agg(h1, src, dst)
    out = _tc_layer2(h1, sum2[:N], sum2[NP:NP + N], cnt[:N], cnt[NP:NP + N],
                     W_l2, W_r2, _pad8(b_l2 + b_r2),
                     W_p1, _pad8(b_p1), W_p2, _pad8(b_p2))
    return out


# speedup vs baseline: 7.4080x; 1.4961x over previous
import jax
import jax.numpy as jnp
from jax import lax
from jax.experimental import pallas as pl
from jax.experimental.pallas import tpu as pltpu
from jax.experimental.pallas import tpu_sc as plsc

N = 10000
E = 320000
D = 128

NC = 2
NS = 16
L = 16
NW = NC * NS

EPW = E // NW
CHUNK = 80
NCHUNK = EPW // CHUNK
NP = 10240
RPT = NP // NS
CW = 16

_MESH = plsc.VectorSubcoreMesh(core_axis_name="c", subcore_axis_name="s")


def _sc_agg_body(x_hbm, src_hbm, dst_hbm, sum_out, sidx_a, sidx_b, didx_a,
                 didx_b, rows_a, rows_b, sum_sh, sem_a, sem_b):
    c = lax.axis_index("c")
    s = lax.axis_index("s")
    wid = s * NC + c
    ebase = wid * EPW


    zero = jnp.zeros((L,), jnp.float32)

    def zrow(i, _):
        for j in range(D // L):
            rows_a[i, pl.ds(j * L, L)] = zero
        return 0

    lax.fori_loop(0, CHUNK, zrow, 0)
    for j in range(RPT // CHUNK):
        pltpu.sync_copy(rows_a, sum_sh.at[pl.ds(s * RPT + j * CHUNK, CHUNK)])
    plsc.subcore_barrier()

    def gather(k, buf, sem, sbuf):
        pltpu.sync_copy(src_hbm.at[pl.ds(ebase + k * CHUNK, CHUNK)], sbuf)
        return pltpu.async_copy(x_hbm.at[sbuf], buf, sem)

    def drain(buf, sem, sbuf):
        pltpu.make_async_copy(x_hbm.at[sbuf], buf, sem).wait()

    def scatter(k, buf, dbuf):
        pltpu.sync_copy(dst_hbm.at[pl.ds(ebase + k * CHUNK, CHUNK)], dbuf)
        pltpu.sync_copy(buf, sum_sh.at[dbuf], add=True)

    gather(0, rows_a, sem_a, sidx_a)
    gather(1, rows_b, sem_b, sidx_b)

    def step(i, _):
        k = i * 2
        drain(rows_a, sem_a, sidx_a)
        scatter(k, rows_a, didx_a)
        gather(k + 2, rows_a, sem_a, sidx_a)
        drain(rows_b, sem_b, sidx_b)
        scatter(k + 1, rows_b, didx_b)
        gather(k + 3, rows_b, sem_b, sidx_b)
        return 0

    lax.fori_loop(0, (NCHUNK - 3) // 2, step, 0)
    drain(rows_a, sem_a, sidx_a)
    scatter(NCHUNK - 3, rows_a, didx_a)
    gather(NCHUNK - 1, rows_a, sem_a, sidx_a)
    drain(rows_b, sem_b, sidx_b)
    scatter(NCHUNK - 2, rows_b, didx_b)
    drain(rows_a, sem_a, sidx_a)
    scatter(NCHUNK - 1, rows_a, didx_a)

    plsc.subcore_barrier()
    pltpu.sync_copy(sum_sh.at[pl.ds(s * RPT, RPT)],
                    sum_out.at[pl.ds(c * NP + s * RPT, RPT)])


_sc_agg = pl.kernel(
    _sc_agg_body,
    out_type=jax.ShapeDtypeStruct((NC * NP, D), jnp.float32),
    mesh=_MESH,
    scratch_types=[
        pltpu.VMEM((CHUNK,), jnp.int32),
        pltpu.VMEM((CHUNK,), jnp.int32),
        pltpu.VMEM((CHUNK,), jnp.int32),
        pltpu.VMEM((CHUNK,), jnp.int32),
        pltpu.VMEM((CHUNK, D), jnp.float32),
        pltpu.VMEM((CHUNK, D), jnp.float32),
        pltpu.VMEM_SHARED((NP, D), jnp.float32),
        pltpu.SemaphoreType.DMA,
        pltpu.SemaphoreType.DMA,
    ],
)

CFIRE = 25


def _sc_cnt_body(dst_hbm, cnt_out, didx, ones, z16, cnt_sh, sem):
    c = lax.axis_index("c")
    s = lax.axis_index("s")
    wid = s * NC + c
    ebase = wid * EPW

    zero = jnp.zeros((L,), jnp.float32)

    def orow(i, _):
        ones[i, :] = zero + 1.0
        return 0

    def zrow(i, _):
        z16[i, :] = zero
        return 0

    lax.fori_loop(0, CHUNK, zrow, 0)
    lax.fori_loop(0, CHUNK, orow, 0)
    for j in range(RPT // CHUNK):
        pltpu.sync_copy(z16, cnt_sh.at[pl.ds(s * RPT + j * CHUNK, CHUNK)])
    plsc.subcore_barrier()

    def chunk_body(k, _):
        pltpu.sync_copy(dst_hbm.at[pl.ds(ebase + k * CHUNK, CHUNK)], didx)
        pltpu.sync_copy(ones, cnt_sh.at[didx], add=True)
        return 0

    lax.fori_loop(0, NCHUNK, chunk_body, 0)

    plsc.subcore_barrier()
    pltpu.sync_copy(cnt_sh.at[pl.ds(s * RPT, RPT)],
                    cnt_out.at[pl.ds(c * NP + s * RPT, RPT)])


_sc_cnt = pl.kernel(
    _sc_cnt_body,
    out_type=jax.ShapeDtypeStruct((NC * NP, CW), jnp.float32),
    mesh=_MESH,
    scratch_types=[
        pltpu.VMEM((CHUNK,), jnp.int32),
        pltpu.VMEM((CHUNK, CW), jnp.float32),
        pltpu.VMEM((CHUNK, CW), jnp.float32),
        pltpu.VMEM_SHARED((NP, CW), jnp.float32),
        pltpu.SemaphoreType.DMA,
    ],
)

BN = 1000


def _tc_layer1_body(x_ref, p0_ref, p1_ref, c0_ref, c1_ref, wl_ref, wr_ref,
                    b_ref, o_ref):
    summed = p0_ref[...] + p1_ref[...]
    cnt = c0_ref[...][:, 0:1] + c1_ref[...][:, 0:1]
    agg = summed / jnp.maximum(cnt, 1.0)
    h = (jnp.dot(x_ref[...], wl_ref[...], preferred_element_type=jnp.float32)
         + jnp.dot(agg, wr_ref[...], preferred_element_type=jnp.float32)
         + b_ref[0:1, :])
    nrm = jnp.sqrt(jnp.sum(h * h, axis=-1, keepdims=True))
    h = h / jnp.maximum(nrm, 1e-12)
    o_ref[...] = jnp.maximum(h, 0.0)


def _tc_layer2_body(x_ref, p0_ref, p1_ref, c0_ref, c1_ref, wl_ref, wr_ref,
                    b_ref, wp1_ref, bp1_ref, wp2_ref, bp2_ref, o_ref):
    summed = p0_ref[...] + p1_ref[...]
    cnt = c0_ref[...][:, 0:1] + c1_ref[...][:, 0:1]
    agg = summed / jnp.maximum(cnt, 1.0)
    h = (jnp.dot(x_ref[...], wl_ref[...], preferred_element_type=jnp.float32)
         + jnp.dot(agg, wr_ref[...], preferred_element_type=jnp.float32)
         + b_ref[0:1, :])
    nrm = jnp.sqrt(jnp.sum(h * h, axis=-1, keepdims=True))
    h = h / jnp.maximum(nrm, 1e-12)
    h = jnp.maximum(h, 0.0)
    g = jnp.maximum(
        jnp.dot(h, wp1_ref[...], preferred_element_type=jnp.float32)
        + bp1_ref[0:1, :], 0.0)
    o_ref[...] = (jnp.dot(g, wp2_ref[...], preferred_element_type=jnp.float32)
                  + bp2_ref[0:1, :])


def _row_spec(width):
    return pl.BlockSpec((BN, width), lambda i: (i, 0))


def _full_spec(r, cdim):
    return pl.BlockSpec((r, cdim), lambda i: (0, 0))


def _tc_layer1(x, p0, p1, c0, c1, wl, wr, bias):
    return pl.pallas_call(
        _tc_layer1_body,
        grid=(N // BN,),
        in_specs=[_row_spec(D), _row_spec(D), _row_spec(D), _row_spec(CW),
                  _row_spec(CW), _full_spec(D, D), _full_spec(D, D),
                  _full_spec(8, D)],
        out_specs=_row_spec(D),
        out_shape=jax.ShapeDtypeStruct((N, D), jnp.float32),
    )(x, p0, p1, c0, c1, wl, wr, bias)


def _tc_layer2(x, p0, p1, c0, c1, wl, wr, bias, wp1, bp1, wp2, bp2):
    return pl.pallas_call(
        _tc_layer2_body,
        grid=(N // BN,),
        in_specs=[_row_spec(D), _row_spec(D), _row_spec(D), _row_spec(CW),
                  _row_spec(CW), _full_spec(D, D), _full_spec(D, D),
                  _full_spec(8, D), _full_spec(D, D), _full_spec(8, D),
                  _full_spec(D, D), _full_spec(8, D)],
        out_specs=_row_spec(D),
        out_shape=jax.ShapeDtypeStruct((N, D), jnp.float32),
    )(x, p0, p1, c0, c1, wl, wr, bias, wp1, bp1, wp2, bp2)


def _pad8(b):
    return jnp.broadcast_to(b[None, :], (8, b.shape[0]))


def kernel(node_features, edge_index, W_l1, b_l1, W_r1, b_r1, W_l2, b_l2,
           W_r2, b_r2, W_p1, b_p1, W_p2, b_p2):
    src = edge_index[0]
    dst = edge_index[1]

    cnt = _sc_cnt(dst)
    sum1 = _sc_agg(node_features, src, dst)
    h1 = _tc_layer1(node_features, sum1[:N], sum1[NP:NP + N], cnt[:N],
                    cnt[NP:NP + N], W_l1, W_r1, _pad8(b_l1 + b_r1))

    sum2 = _sc_agg(h1, src, dst)
    out = _tc_layer2(h1, sum2[:N], sum2[NP:NP + N], cnt[:N], cnt[NP:NP + N],
                     W_l2, W_r2, _pad8(b_l2 + b_r2),
                     W_p1, _pad8(b_p1), W_p2, _pad8(b_p2))
    return out
